# speedup vs baseline: 2.5161x; 2.5161x over previous
import functools

import jax
import jax.numpy as jnp
from jax import lax
from jax.experimental import pallas as pl
from jax.experimental.pallas import tpu as pltpu
from jax.experimental.pallas import tpu_sc as plsc

B = 16384
D = 128
NC = 2
NS = 16
NW = NC * NS
BPW = B // NW
CH = 128
NCH = BPW // CH


def _sc_gather_body(uidx_hbm, iidx_hbm, um_hbm, im_hbm, ug_hbm, ig_hbm,
                    out_u, out_i, out_g,
                    idx_u, idx_i, buf_um, buf_im, buf_ug, buf_ig,
                    s0, s1, s2, s3):
    wid = lax.axis_index("s") * NC + lax.axis_index("c")
    pltpu.sync_copy(uidx_hbm.at[wid], idx_u)
    pltpu.sync_copy(iidx_hbm.at[wid], idx_i)
    for j in range(NCH):
        base = wid * BPW + j * CH
        cp0 = pltpu.async_copy(um_hbm.at[idx_u.at[j]], buf_um, s0)
        cp1 = pltpu.async_copy(im_hbm.at[idx_i.at[j]], buf_im, s1)
        cp2 = pltpu.async_copy(ug_hbm.at[idx_u.at[j]], buf_ug, s2)
        cp3 = pltpu.async_copy(ig_hbm.at[idx_i.at[j]], buf_ig, s3)
        cp0.wait()
        pltpu.sync_copy(buf_um, out_u.at[pl.ds(base, CH)])
        cp1.wait()
        pltpu.sync_copy(buf_im, out_i.at[pl.ds(base, CH)])
        cp2.wait()
        cp3.wait()

        def mul_row(r, carry):
            for k in range(D // 16):
                sl = pl.ds(k * 16, 16)
                buf_ug[r, sl] = buf_ug[r, sl] * buf_ig[r, sl]
            return carry

        lax.fori_loop(0, CH, mul_row, 0)
        pltpu.sync_copy(buf_ug, out_g.at[pl.ds(base, CH)])


_sc_gather = functools.partial(
    pl.kernel,
    out_type=[jax.ShapeDtypeStruct((B, D), jnp.float32)] * 3,
    mesh=plsc.VectorSubcoreMesh(core_axis_name="c", subcore_axis_name="s"),
    scratch_types=[
        pltpu.VMEM((NCH, CH), jnp.int32),
        pltpu.VMEM((NCH, CH), jnp.int32),
        pltpu.VMEM((CH, D), jnp.float32),
        pltpu.VMEM((CH, D), jnp.float32),
        pltpu.VMEM((CH, D), jnp.float32),
        pltpu.VMEM((CH, D), jnp.float32),
        pltpu.SemaphoreType.DMA,
        pltpu.SemaphoreType.DMA,
        pltpu.SemaphoreType.DMA,
        pltpu.SemaphoreType.DMA,
    ],
)(_sc_gather_body)


BLK = 2048


def _mlp_body(u_ref, i_ref, g_ref, w0u_ref, w0i_ref, b0_ref, w1_ref, b1_ref,
              w2_ref, b2_ref, womlp_ref, wogmf_ref, bo_ref, out_ref):
    h = u_ref[...] @ w0u_ref[...] + i_ref[...] @ w0i_ref[...] + b0_ref[...]
    h = jnp.maximum(h, 0.0)
    h = jnp.maximum(h @ w1_ref[...] + b1_ref[...], 0.0)
    h = jnp.maximum(h @ w2_ref[...] + b2_ref[...], 0.0)
    logit = (h @ womlp_ref[...]
             + jnp.sum(g_ref[...] * wogmf_ref[...], axis=1, keepdims=True)
             + bo_ref[...])
    out_ref[...] = jax.nn.sigmoid(logit)


def kernel(user_idx, item_idx, emb_user_mlp, emb_item_mlp, emb_user_gmf,
           emb_item_gmf, W0, b0, W1, b1, W2, b2, W_out, b_out):
    uidx = user_idx.astype(jnp.int32).reshape(NW, NCH, CH)
    iidx = item_idx.astype(jnp.int32).reshape(NW, NCH, CH)
    u_mlp, i_mlp, gmf = _sc_gather(uidx, iidx, emb_user_mlp, emb_item_mlp,
                                   emb_user_gmf, emb_item_gmf)

    w0u = W0[:, :D].T
    w0i = W0[:, D:].T
    w1t = W1.T
    w2t = W2.T
    womlp = W_out[:, :64].T
    wogmf = W_out[:, 64:]
    b0r = b0.reshape(1, -1)
    b1r = b1.reshape(1, -1)
    b2r = b2.reshape(1, -1)
    bor = b_out.reshape(1, 1)

    full = lambda shape: pl.BlockSpec(shape, lambda i: (0, 0))
    rating = pl.pallas_call(
        _mlp_body,
        grid=(B // BLK,),
        in_specs=[
            pl.BlockSpec((BLK, D), lambda i: (i, 0)),
            pl.BlockSpec((BLK, D), lambda i: (i, 0)),
            pl.BlockSpec((BLK, D), lambda i: (i, 0)),
            full((D, 256)), full((D, 256)), full((1, 256)),
            full((256, D)), full((1, D)),
            full((D, 64)), full((1, 64)),
            full((64, 1)), full((1, D)), full((1, 1)),
        ],
        out_specs=pl.BlockSpec((BLK, 1), lambda i: (i, 0)),
        out_shape=jax.ShapeDtypeStruct((B, 1), jnp.float32),
    )(u_mlp, i_mlp, gmf, w0u, w0i, b0r, w1t, b1r, w2t, b2r, womlp, wogmf, bor)
    return rating
